# Initial kernel scaffold; baseline (speedup 1.0000x reference)
import jax
import jax.numpy as jnp
from jax.experimental import pallas as pl


def kernel(x, edge_index, W1, att_src1, att_dst1, b1, W2, att_src2, att_dst2, b2):
    raise NotImplementedError("write your pallas kernel here")


# speedup vs baseline: 18.8944x; 18.8944x over previous
import jax
import jax.numpy as jnp
from jax import lax
from jax.experimental import pallas as pl
from jax.experimental.pallas import tpu as pltpu
from jax.experimental.pallas import tpu_sc as plsc

N = 10000
E = 320000
D = 128
DH = 64
DP = 80
NC = 2
NS = 16
EW = E // NS
K = 80
NCHUNK = EW // K
RSUB = 624
RTAIL = N - NS * RSUB


def _prep_body(z_ref, w_ref, atts_ref, attd_ref, hpads_ref, as_ref, ad_ref,
               sw_ref):
    h = jnp.dot(z_ref[...], w_ref[...], preferred_element_type=jnp.float32)
    a_s = jnp.dot(h, atts_ref[...], preferred_element_type=jnp.float32)
    a_d = jnp.dot(h, attd_ref[...], preferred_element_type=jnp.float32)
    r = h.shape[0]
    col = lax.broadcasted_iota(jnp.int32, (r, DP - DH), 1)
    onespad = jnp.where(col == 0, 1.0, 0.0)
    hpads_ref[0] = jnp.concatenate([h[:, :DH], onespad], axis=1)
    hpads_ref[1] = jnp.concatenate([h[:, DH:], onespad], axis=1)
    as_ref[...] = a_s
    ad_ref[...] = a_d
    s = a_s + a_d
    sw_ref[...] = jnp.exp(jnp.maximum(s, 0.2 * s))


def _tc_prep(z, W, att_s, att_d):
    R = 1000
    grid = N // R
    return pl.pallas_call(
        _prep_body,
        grid=(grid,),
        in_specs=[
            pl.BlockSpec((R, D), lambda i: (i, 0)),
            pl.BlockSpec((D, D), lambda i: (0, 0)),
            pl.BlockSpec((D, 1), lambda i: (0, 0)),
            pl.BlockSpec((D, 1), lambda i: (0, 0)),
        ],
        out_specs=[
            pl.BlockSpec((NC, R, DP), lambda i: (0, i, 0)),
            pl.BlockSpec((R, 1), lambda i: (i, 0)),
            pl.BlockSpec((R, 1), lambda i: (i, 0)),
            pl.BlockSpec((R, 1), lambda i: (i, 0)),
        ],
        out_shape=[
            jax.ShapeDtypeStruct((NC, N, DP), jnp.float32),
            jax.ShapeDtypeStruct((N, 1), jnp.float32),
            jax.ShapeDtypeStruct((N, 1), jnp.float32),
            jax.ShapeDtypeStruct((N, 1), jnp.float32),
        ],
    )(z, W, att_s, att_d)


def _combine_body(acc0_ref, acc1_ref, sw_ref, hpads_ref, b_ref, relu_ref,
                  out_ref):
    h = jnp.concatenate([hpads_ref[0, :, :DH], hpads_ref[1, :, :DH]], axis=1)
    sw = sw_ref[...]
    num = (jnp.concatenate([acc0_ref[:, :DH], acc1_ref[:, :DH]], axis=1)
           + sw * h)
    den = acc0_ref[:, DH:DH + 1] + sw
    res = num / den + b_ref[...]
    out_ref[...] = jnp.where(relu_ref[0] > 0, jnp.maximum(res, 0.0), res)


def _tc_combine(acc0, acc1, sw, hpads, b, relu):
    R = 1000
    grid = N // R
    relu_arr = jnp.full((1,), 1.0 if relu else 0.0, jnp.float32)
    return pl.pallas_call(
        _combine_body,
        grid=(grid,),
        in_specs=[
            pl.BlockSpec((R, DP), lambda i: (i, 0)),
            pl.BlockSpec((R, DP), lambda i: (i, 0)),
            pl.BlockSpec((R, 1), lambda i: (i, 0)),
            pl.BlockSpec((NC, R, DP), lambda i: (0, i, 0)),
            pl.BlockSpec((1, D), lambda i: (0, 0)),
            pl.BlockSpec(memory_space=pltpu.SMEM),
        ],
        out_specs=pl.BlockSpec((R, D), lambda i: (i, 0)),
        out_shape=jax.ShapeDtypeStruct((N, D), jnp.float32),
    )(acc0, acc1, sw, hpads, b, relu_arr)


def _sc_body(hpads_hbm, as_hbm, ad_hbm, src_hbm, dst_hbm, zeros_hbm, acc_out,
             srcv, dstv, astab, adtab, wbuf, rows, acc_sp, gsem):
    c = lax.axis_index("c")
    s = lax.axis_index("s")

    pltpu.sync_copy(src_hbm.at[s], srcv)
    pltpu.sync_copy(dst_hbm.at[s], dstv)
    pltpu.sync_copy(as_hbm, astab)
    pltpu.sync_copy(ad_hbm, adtab)
    pltpu.sync_copy(zeros_hbm, acc_sp.at[pl.ds(s * RSUB, RSUB)])

    @pl.when(s == NS - 1)
    def _():
        pltpu.sync_copy(zeros_hbm.at[pl.ds(0, RTAIL)],
                        acc_sp.at[pl.ds(NS * RSUB, RTAIL)])

    plsc.subcore_barrier()
    htab = hpads_hbm.at[c]

    def chunk(i, carry):
        gather = pltpu.async_copy(htab.at[srcv.at[i]], rows, gsem)
        for j in range(K // 16):
            sidx = srcv[i, pl.ds(j * 16, 16)]
            didx = dstv[i, pl.ds(j * 16, 16)]
            a = plsc.load_gather(astab, [sidx]) + plsc.load_gather(adtab, [didx])
            wbuf[pl.ds(j * 16, 16)] = jnp.exp(jnp.maximum(a, 0.2 * a))
        gather.wait()

        def scale(k, carry2):
            wv = plsc.load_gather(wbuf, [jnp.full((16,), k, jnp.int32)])
            for f in range(DP // 16):
                rows[k, pl.ds(f * 16, 16)] = rows[k, pl.ds(f * 16, 16)] * wv
            return carry2

        lax.fori_loop(0, K, scale, 0)
        pltpu.sync_copy(rows, acc_sp.at[dstv.at[i]], add=True)
        return carry

    lax.fori_loop(0, NCHUNK, chunk, 0)
    plsc.subcore_barrier()
    pltpu.sync_copy(acc_sp.at[pl.ds(s * RSUB, RSUB)],
                    acc_out.at[c, pl.ds(s * RSUB, RSUB)])

    @pl.when(s == NS - 1)
    def _():
        pltpu.sync_copy(acc_sp.at[pl.ds(NS * RSUB, RTAIL)],
                        acc_out.at[c, pl.ds(NS * RSUB, RTAIL)])


def _sc_edge(hpads, a_s, a_d, src, dst, zeros_blk):
    mesh = plsc.VectorSubcoreMesh(core_axis_name="c", subcore_axis_name="s",
                                  num_cores=NC, num_subcores=NS)
    f = pl.kernel(
        _sc_body,
        out_type=jax.ShapeDtypeStruct((NC, N, DP), jnp.float32),
        mesh=mesh,
        compiler_params=pltpu.CompilerParams(needs_layout_passes=False,
                                             use_tc_tiling_on_sc=False),
        scratch_types=[
            pltpu.VMEM((NCHUNK, K), jnp.int32),
            pltpu.VMEM((NCHUNK, K), jnp.int32),
            pltpu.VMEM((N,), jnp.float32),
            pltpu.VMEM((N,), jnp.float32),
            pltpu.VMEM((K,), jnp.float32),
            pltpu.VMEM((K, DP), jnp.float32),
            pltpu.VMEM_SHARED((N, DP), jnp.float32),
            pltpu.SemaphoreType.DMA,
        ],
    )
    return f(hpads, a_s, a_d, src, dst, zeros_blk)


def kernel(x, edge_index, W1, att_src1, att_dst1, b1, W2, att_src2, att_dst2,
           b2):
    src = edge_index[0].reshape(NS, NCHUNK, K)
    dst = edge_index[1].reshape(NS, NCHUNK, K)
    zeros_blk = jnp.zeros((RSUB, DP), jnp.float32)

    def layer(z, W, att_s, att_d, b, relu):
        hpads, a_s, a_d, sw = _tc_prep(z, W, att_s.reshape(D, 1),
                                       att_d.reshape(D, 1))
        acc = _sc_edge(hpads, a_s.reshape(N), a_d.reshape(N), src, dst,
                       zeros_blk)
        return _tc_combine(acc[0], acc[1], sw, hpads, b.reshape(1, D), relu)

    h = layer(x, W1, att_src1, att_dst1, b1, True)
    return layer(h, W2, att_src2, att_dst2, b2, False)


# speedup vs baseline: 31.7069x; 1.6781x over previous
import jax
import jax.numpy as jnp
from jax import lax
from jax.experimental import pallas as pl
from jax.experimental.pallas import tpu as pltpu
from jax.experimental.pallas import tpu_sc as plsc

N = 10000
E = 320000
D = 128
DH = 64
DP = 80
NC = 2
NS = 16
EW = E // NS
K = 80
NCHUNK = EW // K
RSUB = 624
RTAIL = N - NS * RSUB


def _prep_body(z_ref, w_ref, atts_ref, attd_ref, hpads_ref, as_ref, ad_ref,
               sw_ref):
    h = jnp.dot(z_ref[...], w_ref[...], preferred_element_type=jnp.float32)
    a_s = jnp.dot(h, atts_ref[...], preferred_element_type=jnp.float32)
    a_d = jnp.dot(h, attd_ref[...], preferred_element_type=jnp.float32)
    r = h.shape[0]
    col = lax.broadcasted_iota(jnp.int32, (r, DP - DH), 1)
    onespad = jnp.where(col == 0, 1.0, 0.0)
    hpads_ref[0] = jnp.concatenate([h[:, :DH], onespad], axis=1)
    hpads_ref[1] = jnp.concatenate([h[:, DH:], onespad], axis=1)
    as_ref[...] = a_s
    ad_ref[...] = a_d
    s = a_s + a_d
    sw_ref[...] = jnp.exp(jnp.maximum(s, 0.2 * s))


def _tc_prep(z, W, att_s, att_d):
    R = 1000
    grid = N // R
    return pl.pallas_call(
        _prep_body,
        grid=(grid,),
        in_specs=[
            pl.BlockSpec((R, D), lambda i: (i, 0)),
            pl.BlockSpec((D, D), lambda i: (0, 0)),
            pl.BlockSpec((D, 1), lambda i: (0, 0)),
            pl.BlockSpec((D, 1), lambda i: (0, 0)),
        ],
        out_specs=[
            pl.BlockSpec((NC, R, DP), lambda i: (0, i, 0)),
            pl.BlockSpec((R, 1), lambda i: (i, 0)),
            pl.BlockSpec((R, 1), lambda i: (i, 0)),
            pl.BlockSpec((R, 1), lambda i: (i, 0)),
        ],
        out_shape=[
            jax.ShapeDtypeStruct((NC, N, DP), jnp.float32),
            jax.ShapeDtypeStruct((N, 1), jnp.float32),
            jax.ShapeDtypeStruct((N, 1), jnp.float32),
            jax.ShapeDtypeStruct((N, 1), jnp.float32),
        ],
    )(z, W, att_s, att_d)


def _combine_body(acc0_ref, acc1_ref, sw_ref, hpads_ref, b_ref, relu_ref,
                  out_ref):
    h = jnp.concatenate([hpads_ref[0, :, :DH], hpads_ref[1, :, :DH]], axis=1)
    sw = sw_ref[...]
    num = (jnp.concatenate([acc0_ref[:, :DH], acc1_ref[:, :DH]], axis=1)
           + sw * h)
    den = acc0_ref[:, DH:DH + 1] + sw
    res = num / den + b_ref[...]
    out_ref[...] = jnp.where(relu_ref[0] > 0, jnp.maximum(res, 0.0), res)


def _tc_combine(acc0, acc1, sw, hpads, b, relu):
    R = 1000
    grid = N // R
    relu_arr = jnp.full((1,), 1.0 if relu else 0.0, jnp.float32)
    return pl.pallas_call(
        _combine_body,
        grid=(grid,),
        in_specs=[
            pl.BlockSpec((R, DP), lambda i: (i, 0)),
            pl.BlockSpec((R, DP), lambda i: (i, 0)),
            pl.BlockSpec((R, 1), lambda i: (i, 0)),
            pl.BlockSpec((NC, R, DP), lambda i: (0, i, 0)),
            pl.BlockSpec((1, D), lambda i: (0, 0)),
            pl.BlockSpec(memory_space=pltpu.SMEM),
        ],
        out_specs=pl.BlockSpec((R, D), lambda i: (i, 0)),
        out_shape=jax.ShapeDtypeStruct((N, D), jnp.float32),
    )(acc0, acc1, sw, hpads, b, relu_arr)


def _sc_body(hpads_hbm, as_hbm, ad_hbm, src_hbm, dst_hbm, zeros_hbm, acc_out,
             srcv, dstv, astab, adtab, wbuf, rows, acc_sp, gsem, ssem):
    c = lax.axis_index("c")
    s = lax.axis_index("s")

    pltpu.sync_copy(src_hbm.at[s], srcv)
    pltpu.sync_copy(dst_hbm.at[s], dstv)
    pltpu.sync_copy(as_hbm, astab)
    pltpu.sync_copy(ad_hbm, adtab)
    pltpu.sync_copy(zeros_hbm, acc_sp.at[pl.ds(s * RSUB, RSUB)])

    @pl.when(s == NS - 1)
    def _():
        pltpu.sync_copy(zeros_hbm.at[pl.ds(0, RTAIL)],
                        acc_sp.at[pl.ds(NS * RSUB, RTAIL)])

    plsc.subcore_barrier()
    htab = hpads_hbm.at[c]

    pltpu.async_copy(htab.at[srcv.at[0]], rows.at[0], gsem)

    def chunk(i, carry):
        b = lax.rem(i, 2)

        for j in range(K // 16):
            sidx = srcv[i, pl.ds(j * 16, 16)]
            didx = dstv[i, pl.ds(j * 16, 16)]
            a = plsc.load_gather(astab, [sidx]) + plsc.load_gather(adtab, [didx])
            wbuf[pl.ds(j * 16, 16)] = jnp.exp(jnp.maximum(a, 0.2 * a))

        @pl.when(i > 0)
        def _():
            pltpu.make_async_copy(rows.at[1 - b], acc_sp.at[dstv.at[i - 1]],
                                  ssem).wait()

        @pl.when(i < NCHUNK - 1)
        def _():
            pltpu.async_copy(htab.at[srcv.at[i + 1]], rows.at[1 - b], gsem)

        pltpu.make_async_copy(htab.at[srcv.at[i]], rows.at[b], gsem).wait()

        @plsc.parallel_loop(0, K, unroll=8)
        def scale(k):
            wv = plsc.load_gather(wbuf, [jnp.full((16,), k, jnp.int32)])
            for f in range(DP // 16):
                rows[b, k, pl.ds(f * 16, 16)] = rows[b, k, pl.ds(f * 16, 16)] * wv

        pltpu.async_copy(rows.at[b], acc_sp.at[dstv.at[i]], ssem, add=True)
        return carry

    lax.fori_loop(0, NCHUNK, chunk, 0)
    pltpu.make_async_copy(rows.at[lax.rem(NCHUNK - 1, 2)],
                          acc_sp.at[dstv.at[NCHUNK - 1]], ssem).wait()
    plsc.subcore_barrier()
    pltpu.sync_copy(acc_sp.at[pl.ds(s * RSUB, RSUB)],
                    acc_out.at[c, pl.ds(s * RSUB, RSUB)])

    @pl.when(s == NS - 1)
    def _():
        pltpu.sync_copy(acc_sp.at[pl.ds(NS * RSUB, RTAIL)],
                        acc_out.at[c, pl.ds(NS * RSUB, RTAIL)])


def _sc_edge(hpads, a_s, a_d, src, dst, zeros_blk):
    mesh = plsc.VectorSubcoreMesh(core_axis_name="c", subcore_axis_name="s",
                                  num_cores=NC, num_subcores=NS)
    f = pl.kernel(
        _sc_body,
        out_type=jax.ShapeDtypeStruct((NC, N, DP), jnp.float32),
        mesh=mesh,
        compiler_params=pltpu.CompilerParams(needs_layout_passes=False,
                                             use_tc_tiling_on_sc=False),
        scratch_types=[
            pltpu.VMEM((NCHUNK, K), jnp.int32),
            pltpu.VMEM((NCHUNK, K), jnp.int32),
            pltpu.VMEM((N,), jnp.float32),
            pltpu.VMEM((N,), jnp.float32),
            pltpu.VMEM((K,), jnp.float32),
            pltpu.VMEM((2, K, DP), jnp.float32),
            pltpu.VMEM_SHARED((N, DP), jnp.float32),
            pltpu.SemaphoreType.DMA,
            pltpu.SemaphoreType.DMA,
        ],
    )
    return f(hpads, a_s, a_d, src, dst, zeros_blk)


def kernel(x, edge_index, W1, att_src1, att_dst1, b1, W2, att_src2, att_dst2,
           b2):
    src = edge_index[0].reshape(NS, NCHUNK, K)
    dst = edge_index[1].reshape(NS, NCHUNK, K)
    zeros_blk = jnp.zeros((RSUB, DP), jnp.float32)

    def layer(z, W, att_s, att_d, b, relu):
        hpads, a_s, a_d, sw = _tc_prep(z, W, att_s.reshape(D, 1),
                                       att_d.reshape(D, 1))
        acc = _sc_edge(hpads, a_s.reshape(N), a_d.reshape(N), src, dst,
                       zeros_blk)
        return _tc_combine(acc[0], acc[1], sw, hpads, b.reshape(1, D), relu)

    h = layer(x, W1, att_src1, att_dst1, b1, True)
    return layer(h, W2, att_src2, att_dst2, b2, False)


# speedup vs baseline: 37.8612x; 1.1941x over previous
import jax
import jax.numpy as jnp
from jax import lax
from jax.experimental import pallas as pl
from jax.experimental.pallas import tpu as pltpu
from jax.experimental.pallas import tpu_sc as plsc

N = 10000
E = 320000
D = 128
DH = 64
DP = 80
NC = 2
NS = 16
EW = E // NS
K = 80
NCHUNK = EW // K
NBUF = 3
RSUB = 624
RTAIL = N - NS * RSUB
R = 1000


def _emit_prep(h, hpads_ref, as_ref, ad_ref, sw_ref, atts_ref, attd_ref):
    a_s = jnp.dot(h, atts_ref[...], preferred_element_type=jnp.float32)
    a_d = jnp.dot(h, attd_ref[...], preferred_element_type=jnp.float32)
    col = lax.broadcasted_iota(jnp.int32, (h.shape[0], DP - DH), 1)
    onespad = jnp.where(col == 0, 1.0, 0.0)
    hpads_ref[0] = jnp.concatenate([h[:, :DH], onespad], axis=1)
    hpads_ref[1] = jnp.concatenate([h[:, DH:], onespad], axis=1)
    as_ref[...] = a_s
    ad_ref[...] = a_d
    s = a_s + a_d
    sw_ref[...] = jnp.exp(jnp.maximum(s, 0.2 * s))


def _combine(acc_ref, sw_ref, hpads_ref, b_ref):
    h = jnp.concatenate([hpads_ref[0, :, :DH], hpads_ref[1, :, :DH]], axis=1)
    sw = sw_ref[...]
    num = (jnp.concatenate([acc_ref[0, :, :DH], acc_ref[1, :, :DH]], axis=1)
           + sw * h)
    den = acc_ref[0, :, DH:DH + 1] + sw
    return num / den + b_ref[...]


def _prep_body(z_ref, w_ref, atts_ref, attd_ref, hpads_ref, as_ref, ad_ref,
               sw_ref):
    h = jnp.dot(z_ref[...], w_ref[...], preferred_element_type=jnp.float32)
    _emit_prep(h, hpads_ref, as_ref, ad_ref, sw_ref, atts_ref, attd_ref)


def _mid_body(acc_ref, sw_ref, hpads_ref, b_ref, w_ref, atts_ref, attd_ref,
              hpads2_ref, as_ref, ad_ref, sw2_ref):
    z = jnp.maximum(_combine(acc_ref, sw_ref, hpads_ref, b_ref), 0.0)
    h2 = jnp.dot(z, w_ref[...], preferred_element_type=jnp.float32)
    _emit_prep(h2, hpads2_ref, as_ref, ad_ref, sw2_ref, atts_ref, attd_ref)


def _fin_body(acc_ref, sw_ref, hpads_ref, b_ref, out_ref):
    out_ref[...] = _combine(acc_ref, sw_ref, hpads_ref, b_ref)


_prep_outs = dict(
    out_specs=[
        pl.BlockSpec((NC, R, DP), lambda i: (0, i, 0)),
        pl.BlockSpec((R, 1), lambda i: (i, 0)),
        pl.BlockSpec((R, 1), lambda i: (i, 0)),
        pl.BlockSpec((R, 1), lambda i: (i, 0)),
    ],
    out_shape=[
        jax.ShapeDtypeStruct((NC, N, DP), jnp.float32),
        jax.ShapeDtypeStruct((N, 1), jnp.float32),
        jax.ShapeDtypeStruct((N, 1), jnp.float32),
        jax.ShapeDtypeStruct((N, 1), jnp.float32),
    ],
)


def _tc_prep(z, W, att_s, att_d):
    return pl.pallas_call(
        _prep_body,
        grid=(N // R,),
        in_specs=[
            pl.BlockSpec((R, D), lambda i: (i, 0)),
            pl.BlockSpec((D, D), lambda i: (0, 0)),
            pl.BlockSpec((D, 1), lambda i: (0, 0)),
            pl.BlockSpec((D, 1), lambda i: (0, 0)),
        ],
        **_prep_outs,
    )(z, W, att_s, att_d)


def _tc_mid(acc, sw, hpads, b, W, att_s, att_d):
    return pl.pallas_call(
        _mid_body,
        grid=(N // R,),
        in_specs=[
            pl.BlockSpec((NC, R, DP), lambda i: (0, i, 0)),
            pl.BlockSpec((R, 1), lambda i: (i, 0)),
            pl.BlockSpec((NC, R, DP), lambda i: (0, i, 0)),
            pl.BlockSpec((1, D), lambda i: (0, 0)),
            pl.BlockSpec((D, D), lambda i: (0, 0)),
            pl.BlockSpec((D, 1), lambda i: (0, 0)),
            pl.BlockSpec((D, 1), lambda i: (0, 0)),
        ],
        **_prep_outs,
    )(acc, sw, hpads, b, W, att_s, att_d)


def _tc_fin(acc, sw, hpads, b):
    return pl.pallas_call(
        _fin_body,
        grid=(N // R,),
        in_specs=[
            pl.BlockSpec((NC, R, DP), lambda i: (0, i, 0)),
            pl.BlockSpec((R, 1), lambda i: (i, 0)),
            pl.BlockSpec((NC, R, DP), lambda i: (0, i, 0)),
            pl.BlockSpec((1, D), lambda i: (0, 0)),
        ],
        out_specs=pl.BlockSpec((R, D), lambda i: (i, 0)),
        out_shape=jax.ShapeDtypeStruct((N, D), jnp.float32),
    )(acc, sw, hpads, b)


def _sc_body(hpads_hbm, as_hbm, ad_hbm, src_hbm, dst_hbm, zeros_hbm, acc_out,
             srcv, dstv, astab, adtab, wbuf, rows, acc_sp, gsem, ssem):
    c = lax.axis_index("c")
    s = lax.axis_index("s")

    pltpu.sync_copy(src_hbm.at[s], srcv)
    pltpu.sync_copy(dst_hbm.at[s], dstv)
    pltpu.sync_copy(as_hbm, astab)
    pltpu.sync_copy(ad_hbm, adtab)
    pltpu.sync_copy(zeros_hbm, acc_sp.at[pl.ds(s * RSUB, RSUB)])

    @pl.when(s == NS - 1)
    def _():
        pltpu.sync_copy(zeros_hbm.at[pl.ds(0, RTAIL)],
                        acc_sp.at[pl.ds(NS * RSUB, RTAIL)])

    plsc.subcore_barrier()
    htab = hpads_hbm.at[c]

    pltpu.async_copy(htab.at[srcv.at[0]], rows.at[0], gsem)

    def chunk(i, carry):
        b = lax.rem(i, NBUF)
        bn = lax.rem(i + 1, NBUF)

        for j in range(K // 16):
            sidx = srcv[i, pl.ds(j * 16, 16)]
            didx = dstv[i, pl.ds(j * 16, 16)]
            a = plsc.load_gather(astab, [sidx]) + plsc.load_gather(adtab, [didx])
            wbuf[pl.ds(j * 16, 16)] = jnp.exp(jnp.maximum(a, 0.2 * a))

        @pl.when(i >= NBUF - 1)
        def _():
            pltpu.make_async_copy(rows.at[bn],
                                  acc_sp.at[dstv.at[i - (NBUF - 1)]],
                                  ssem).wait()

        @pl.when(i < NCHUNK - 1)
        def _():
            pltpu.async_copy(htab.at[srcv.at[i + 1]], rows.at[bn], gsem)

        pltpu.make_async_copy(htab.at[srcv.at[i]], rows.at[b], gsem).wait()

        @plsc.parallel_loop(0, K, unroll=8)
        def scale(k):
            wv = plsc.load_gather(wbuf, [jnp.full((16,), k, jnp.int32)])
            for f in range(DP // 16):
                rows[b, k, pl.ds(f * 16, 16)] = rows[b, k, pl.ds(f * 16, 16)] * wv

        pltpu.async_copy(rows.at[b], acc_sp.at[dstv.at[i]], ssem, add=True)
        return carry

    lax.fori_loop(0, NCHUNK, chunk, 0)
    for t in range(NBUF - 1):
        i = NCHUNK - (NBUF - 1) + t
        pltpu.make_async_copy(rows.at[lax.rem(i, NBUF)],
                              acc_sp.at[dstv.at[i]], ssem).wait()
    plsc.subcore_barrier()
    pltpu.sync_copy(acc_sp.at[pl.ds(s * RSUB, RSUB)],
                    acc_out.at[c, pl.ds(s * RSUB, RSUB)])

    @pl.when(s == NS - 1)
    def _():
        pltpu.sync_copy(acc_sp.at[pl.ds(NS * RSUB, RTAIL)],
                        acc_out.at[c, pl.ds(NS * RSUB, RTAIL)])


def _sc_edge(hpads, a_s, a_d, src, dst, zeros_blk):
    mesh = plsc.VectorSubcoreMesh(core_axis_name="c", subcore_axis_name="s",
                                  num_cores=NC, num_subcores=NS)
    f = pl.kernel(
        _sc_body,
        out_type=jax.ShapeDtypeStruct((NC, N, DP), jnp.float32),
        mesh=mesh,
        compiler_params=pltpu.CompilerParams(needs_layout_passes=False,
                                             use_tc_tiling_on_sc=False),
        scratch_types=[
            pltpu.VMEM((NCHUNK, K), jnp.int32),
            pltpu.VMEM((NCHUNK, K), jnp.int32),
            pltpu.VMEM((N,), jnp.float32),
            pltpu.VMEM((N,), jnp.float32),
            pltpu.VMEM((K,), jnp.float32),
            pltpu.VMEM((NBUF, K, DP), jnp.float32),
            pltpu.VMEM_SHARED((N, DP), jnp.float32),
            pltpu.SemaphoreType.DMA,
            pltpu.SemaphoreType.DMA,
        ],
    )
    return f(hpads, a_s, a_d, src, dst, zeros_blk)


def kernel(x, edge_index, W1, att_src1, att_dst1, b1, W2, att_src2, att_dst2,
           b2):
    src = edge_index[0].reshape(NS, NCHUNK, K)
    dst = edge_index[1].reshape(NS, NCHUNK, K)
    zeros_blk = jnp.zeros((RSUB, DP), jnp.float32)

    hpads1, a_s1, a_d1, sw1 = _tc_prep(x, W1, att_src1.reshape(D, 1),
                                       att_dst1.reshape(D, 1))
    acc1 = _sc_edge(hpads1, a_s1.reshape(N), a_d1.reshape(N), src, dst,
                    zeros_blk)
    hpads2, a_s2, a_d2, sw2 = _tc_mid(acc1, sw1, hpads1, b1.reshape(1, D),
                                      W2, att_src2.reshape(D, 1),
                                      att_dst2.reshape(D, 1))
    acc2 = _sc_edge(hpads2, a_s2.reshape(N), a_d2.reshape(N), src, dst,
                    zeros_blk)
    return _tc_fin(acc2, sw2, hpads2, b2.reshape(1, D))


# speedup vs baseline: 42.1245x; 1.1126x over previous
import jax
import jax.numpy as jnp
from jax import lax
from jax.experimental import pallas as pl
from jax.experimental.pallas import tpu as pltpu
from jax.experimental.pallas import tpu_sc as plsc

N = 10000
E = 320000
D = 128
DH = 64
DW = 8
NC = 2
NS = 16
EW = E // NS
K = 80
NCHUNK = EW // K
NBUF = 4
RSUB = 624
RTAIL = N - NS * RSUB
R = 1000


def _emit_prep(h, hpads_ref, as_ref, ad_ref, sw_ref, atts_ref, attd_ref):
    a_s = jnp.dot(h, atts_ref[...], preferred_element_type=jnp.float32)
    a_d = jnp.dot(h, attd_ref[...], preferred_element_type=jnp.float32)
    hpads_ref[0] = h[:, :DH]
    hpads_ref[1] = h[:, DH:]
    as_ref[...] = a_s
    ad_ref[...] = a_d
    s = a_s + a_d
    sw_ref[...] = jnp.exp(jnp.maximum(s, 0.2 * s))


def _combine(acc_ref, den_ref, sw_ref, hpads_ref, b_ref):
    h = jnp.concatenate([hpads_ref[0], hpads_ref[1]], axis=1)
    sw = sw_ref[...]
    num = jnp.concatenate([acc_ref[0], acc_ref[1]], axis=1) + sw * h
    den = den_ref[0, :, 0:1] + sw
    return num / den + b_ref[...]


def _prep_body(z_ref, w_ref, atts_ref, attd_ref, hpads_ref, as_ref, ad_ref,
               sw_ref):
    h = jnp.dot(z_ref[...], w_ref[...], preferred_element_type=jnp.float32)
    _emit_prep(h, hpads_ref, as_ref, ad_ref, sw_ref, atts_ref, attd_ref)


def _mid_body(acc_ref, den_ref, sw_ref, hpads_ref, b_ref, w_ref, atts_ref,
              attd_ref, hpads2_ref, as_ref, ad_ref, sw2_ref):
    z = jnp.maximum(_combine(acc_ref, den_ref, sw_ref, hpads_ref, b_ref), 0.0)
    h2 = jnp.dot(z, w_ref[...], preferred_element_type=jnp.float32)
    _emit_prep(h2, hpads2_ref, as_ref, ad_ref, sw2_ref, atts_ref, attd_ref)


def _fin_body(acc_ref, den_ref, sw_ref, hpads_ref, b_ref, out_ref):
    out_ref[...] = _combine(acc_ref, den_ref, sw_ref, hpads_ref, b_ref)


_prep_outs = dict(
    out_specs=[
        pl.BlockSpec((NC, R, DH), lambda i: (0, i, 0)),
        pl.BlockSpec((R, 1), lambda i: (i, 0)),
        pl.BlockSpec((R, 1), lambda i: (i, 0)),
        pl.BlockSpec((R, 1), lambda i: (i, 0)),
    ],
    out_shape=[
        jax.ShapeDtypeStruct((NC, N, DH), jnp.float32),
        jax.ShapeDtypeStruct((N, 1), jnp.float32),
        jax.ShapeDtypeStruct((N, 1), jnp.float32),
        jax.ShapeDtypeStruct((N, 1), jnp.float32),
    ],
)

_acc_specs = [
    pl.BlockSpec((NC, R, DH), lambda i: (0, i, 0)),
    pl.BlockSpec((NC, R, DW), lambda i: (0, i, 0)),
    pl.BlockSpec((R, 1), lambda i: (i, 0)),
    pl.BlockSpec((NC, R, DH), lambda i: (0, i, 0)),
    pl.BlockSpec((1, D), lambda i: (0, 0)),
]


def _tc_prep(z, W, att_s, att_d):
    return pl.pallas_call(
        _prep_body,
        grid=(N // R,),
        in_specs=[
            pl.BlockSpec((R, D), lambda i: (i, 0)),
            pl.BlockSpec((D, D), lambda i: (0, 0)),
            pl.BlockSpec((D, 1), lambda i: (0, 0)),
            pl.BlockSpec((D, 1), lambda i: (0, 0)),
        ],
        **_prep_outs,
    )(z, W, att_s, att_d)


def _tc_mid(acc, den, sw, hpads, b, W, att_s, att_d):
    return pl.pallas_call(
        _mid_body,
        grid=(N // R,),
        in_specs=_acc_specs + [
            pl.BlockSpec((D, D), lambda i: (0, 0)),
            pl.BlockSpec((D, 1), lambda i: (0, 0)),
            pl.BlockSpec((D, 1), lambda i: (0, 0)),
        ],
        **_prep_outs,
    )(acc, den, sw, hpads, b, W, att_s, att_d)


def _tc_fin(acc, den, sw, hpads, b):
    return pl.pallas_call(
        _fin_body,
        grid=(N // R,),
        in_specs=_acc_specs,
        out_specs=pl.BlockSpec((R, D), lambda i: (i, 0)),
        out_shape=jax.ShapeDtypeStruct((N, D), jnp.float32),
    )(acc, den, sw, hpads, b)


def _sc_body(hpads_hbm, as_hbm, ad_hbm, src_hbm, dst_hbm, zeros_hbm, zd_hbm,
             acc_out, den_out,
             srcv, dstv, astab, adtab, wbufs, rows, acc_sp, den_sp,
             gsem, ssem, wsem, psem):
    c = lax.axis_index("c")
    s = lax.axis_index("s")

    pltpu.async_copy(src_hbm.at[s], srcv, psem)
    pltpu.async_copy(dst_hbm.at[s], dstv, psem)
    pltpu.async_copy(as_hbm, astab, psem)
    pltpu.async_copy(ad_hbm, adtab, psem)
    pltpu.async_copy(zeros_hbm, acc_sp.at[pl.ds(s * RSUB, RSUB)], psem)
    pltpu.async_copy(zd_hbm.at[pl.ds(0, RSUB)],
                     den_sp.at[pl.ds(s * RSUB, RSUB)], psem)
    pltpu.async_copy(zd_hbm.at[pl.ds(0, NBUF * K)], wbufs, psem)

    @pl.when(s == NS - 1)
    def _():
        pltpu.async_copy(zeros_hbm.at[pl.ds(0, RTAIL)],
                         acc_sp.at[pl.ds(NS * RSUB, RTAIL)], psem)
        pltpu.async_copy(zd_hbm.at[pl.ds(0, RTAIL)],
                         den_sp.at[pl.ds(NS * RSUB, RTAIL)], psem)

    pltpu.make_async_copy(src_hbm.at[s], srcv, psem).wait()
    pltpu.make_async_copy(dst_hbm.at[s], dstv, psem).wait()
    pltpu.make_async_copy(as_hbm, astab, psem).wait()
    pltpu.make_async_copy(ad_hbm, adtab, psem).wait()
    pltpu.make_async_copy(zeros_hbm, acc_sp.at[pl.ds(s * RSUB, RSUB)],
                          psem).wait()
    pltpu.make_async_copy(zd_hbm.at[pl.ds(0, RSUB)],
                          den_sp.at[pl.ds(s * RSUB, RSUB)], psem).wait()
    pltpu.make_async_copy(zd_hbm.at[pl.ds(0, NBUF * K)], wbufs, psem).wait()

    @pl.when(s == NS - 1)
    def _():
        pltpu.make_async_copy(zeros_hbm.at[pl.ds(0, RTAIL)],
                              acc_sp.at[pl.ds(NS * RSUB, RTAIL)], psem).wait()
        pltpu.make_async_copy(zd_hbm.at[pl.ds(0, RTAIL)],
                              den_sp.at[pl.ds(NS * RSUB, RTAIL)], psem).wait()

    plsc.subcore_barrier()
    htab = hpads_hbm.at[c]
    col0 = jnp.zeros((16,), jnp.int32)
    lane = lax.iota(jnp.int32, 16)

    pltpu.async_copy(htab.at[srcv.at[0]], rows.at[0], gsem)

    def chunk(i, carry):
        b = lax.rem(i, NBUF)
        bn = lax.rem(i + 1, NBUF)

        @pl.when(i >= NBUF)
        def _():
            pltpu.make_async_copy(
                wbufs.at[pl.ds(b * K, K)],
                den_sp.at[dstv.at[i - NBUF]], wsem).wait()

        for j in range(K // 16):
            sidx = srcv[i, pl.ds(j * 16, 16)]
            didx = dstv[i, pl.ds(j * 16, 16)]
            a = plsc.load_gather(astab, [sidx]) + plsc.load_gather(adtab, [didx])
            w16 = jnp.exp(jnp.maximum(a, 0.2 * a))
            plsc.store_scatter(wbufs, [b * K + j * 16 + lane, col0], w16)
        pltpu.async_copy(wbufs.at[pl.ds(b * K, K)],
                         den_sp.at[dstv.at[i]], wsem, add=True)

        @pl.when(i >= NBUF - 1)
        def _():
            pltpu.make_async_copy(rows.at[bn],
                                  acc_sp.at[dstv.at[i - (NBUF - 1)]],
                                  ssem).wait()

        @pl.when(i < NCHUNK - 1)
        def _():
            pltpu.async_copy(htab.at[srcv.at[i + 1]], rows.at[bn], gsem)

        pltpu.make_async_copy(htab.at[srcv.at[i]], rows.at[b], gsem).wait()

        @plsc.parallel_loop(0, K, unroll=8)
        def scale(k):
            wv = plsc.load_gather(wbufs, [jnp.full((16,), b * K + k,
                                                   jnp.int32), col0])
            for f in range(DH // 16):
                rows[b, k, pl.ds(f * 16, 16)] = rows[b, k, pl.ds(f * 16, 16)] * wv

        pltpu.async_copy(rows.at[b], acc_sp.at[dstv.at[i]], ssem, add=True)
        return carry

    lax.fori_loop(0, NCHUNK, chunk, 0)
    for t in range(NBUF - 1):
        i = NCHUNK - (NBUF - 1) + t
        pltpu.make_async_copy(rows.at[lax.rem(i, NBUF)],
                              acc_sp.at[dstv.at[i]], ssem).wait()
    for t in range(NBUF):
        i = NCHUNK - NBUF + t
        pltpu.make_async_copy(wbufs.at[pl.ds(lax.rem(i, NBUF) * K, K)],
                              den_sp.at[dstv.at[i]], wsem).wait()
    plsc.subcore_barrier()
    pltpu.sync_copy(acc_sp.at[pl.ds(s * RSUB, RSUB)],
                    acc_out.at[c, pl.ds(s * RSUB, RSUB)])
    pltpu.sync_copy(den_sp.at[pl.ds(s * RSUB, RSUB)],
                    den_out.at[c, pl.ds(s * RSUB, RSUB)])

    @pl.when(s == NS - 1)
    def _():
        pltpu.sync_copy(acc_sp.at[pl.ds(NS * RSUB, RTAIL)],
                        acc_out.at[c, pl.ds(NS * RSUB, RTAIL)])
        pltpu.sync_copy(den_sp.at[pl.ds(NS * RSUB, RTAIL)],
                        den_out.at[c, pl.ds(NS * RSUB, RTAIL)])


def _sc_edge(hpads, a_s, a_d, src, dst, zeros_blk, zd):
    mesh = plsc.VectorSubcoreMesh(core_axis_name="c", subcore_axis_name="s",
                                  num_cores=NC, num_subcores=NS)
    f = pl.kernel(
        _sc_body,
        out_type=[
            jax.ShapeDtypeStruct((NC, N, DH), jnp.float32),
            jax.ShapeDtypeStruct((NC, N, DW), jnp.float32),
        ],
        mesh=mesh,
        compiler_params=pltpu.CompilerParams(needs_layout_passes=False,
                                             use_tc_tiling_on_sc=False),
        scratch_types=[
            pltpu.VMEM((NCHUNK, K), jnp.int32),
            pltpu.VMEM((NCHUNK, K), jnp.int32),
            pltpu.VMEM((N,), jnp.float32),
            pltpu.VMEM((N,), jnp.float32),
            pltpu.VMEM((NBUF * K, DW), jnp.float32),
            pltpu.VMEM((NBUF, K, DH), jnp.float32),
            pltpu.VMEM_SHARED((N, DH), jnp.float32),
            pltpu.VMEM_SHARED((N, DW), jnp.float32),
            pltpu.SemaphoreType.DMA,
            pltpu.SemaphoreType.DMA,
            pltpu.SemaphoreType.DMA,
            pltpu.SemaphoreType.DMA,
        ],
    )
    return f(hpads, a_s, a_d, src, dst, zeros_blk, zd)


def kernel(x, edge_index, W1, att_src1, att_dst1, b1, W2, att_src2, att_dst2,
           b2):
    src = edge_index[0].reshape(NS, NCHUNK, K)
    dst = edge_index[1].reshape(NS, NCHUNK, K)
    zeros_blk = jnp.zeros((RSUB, DH), jnp.float32)
    zd = jnp.zeros((RSUB, DW), jnp.float32)

    hpads1, a_s1, a_d1, sw1 = _tc_prep(x, W1, att_src1.reshape(D, 1),
                                       att_dst1.reshape(D, 1))
    acc1, den1 = _sc_edge(hpads1, a_s1.reshape(N), a_d1.reshape(N), src, dst,
                          zeros_blk, zd)
    hpads2, a_s2, a_d2, sw2 = _tc_mid(acc1, den1, sw1, hpads1,
                                      b1.reshape(1, D), W2,
                                      att_src2.reshape(D, 1),
                                      att_dst2.reshape(D, 1))
    acc2, den2 = _sc_edge(hpads2, a_s2.reshape(N), a_d2.reshape(N), src, dst,
                          zeros_blk, zd)
    return _tc_fin(acc2, den2, sw2, hpads2, b2.reshape(1, D))


# speedup vs baseline: 45.3211x; 1.0759x over previous
import numpy as np

import jax
import jax.numpy as jnp
from jax import lax
from jax.experimental import pallas as pl
from jax.experimental.pallas import tpu as pltpu
from jax.experimental.pallas import tpu_sc as plsc

N = 10000
E = 320000
D = 128
DH = 64
DW = 8
NC = 2
NS = 16
EW = E // NS
K = 80
NCHUNK = EW // K
NBG = 2
NBS = 3
NBW = 4
RSUB = 624
RTAIL = N - NS * RSUB
R = 1000

_g = np.arange(D) // 32 * 32
_t = np.arange(D) % 32
PERM = (_g + np.where(_t % 2 == 0, _t // 2, 16 + _t // 2)).astype(np.int32)


def _emit_prep(h, hpads_ref, as_ref, ad_ref, sw_ref, atts_ref, attd_ref):
    a_s = jnp.dot(h, atts_ref[...], preferred_element_type=jnp.float32)
    a_d = jnp.dot(h, attd_ref[...], preferred_element_type=jnp.float32)
    hpads_ref[0] = h[:, :DH].astype(jnp.bfloat16)
    hpads_ref[1] = h[:, DH:].astype(jnp.bfloat16)
    as_ref[...] = a_s
    ad_ref[...] = a_d
    s = a_s + a_d
    sw_ref[...] = jnp.exp(jnp.maximum(s, 0.2 * s))


def _combine(acc_ref, den_ref, sw_ref, hpads_ref, pmat_ref, b_ref):
    ht = jnp.concatenate([hpads_ref[0], hpads_ref[1]],
                         axis=1).astype(jnp.float32)
    h = jnp.dot(ht, pmat_ref[...], preferred_element_type=jnp.float32)
    sw = sw_ref[...]
    num = jnp.concatenate([acc_ref[0], acc_ref[1]], axis=1) + sw * h
    den = den_ref[0, :, 0:1] + sw
    return num / den + b_ref[...]


def _prep_body(z_ref, w_ref, atts_ref, attd_ref, hpads_ref, as_ref, ad_ref,
               sw_ref):
    h = jnp.dot(z_ref[...], w_ref[...], preferred_element_type=jnp.float32)
    _emit_prep(h, hpads_ref, as_ref, ad_ref, sw_ref, atts_ref, attd_ref)


def _mid_body(acc_ref, den_ref, sw_ref, hpads_ref, pmat_ref, b_ref, w_ref,
              atts_ref, attd_ref, hpads2_ref, as_ref, ad_ref, sw2_ref):
    z = jnp.maximum(_combine(acc_ref, den_ref, sw_ref, hpads_ref, pmat_ref,
                             b_ref), 0.0)
    h2 = jnp.dot(z, w_ref[...], preferred_element_type=jnp.float32)
    _emit_prep(h2, hpads2_ref, as_ref, ad_ref, sw2_ref, atts_ref, attd_ref)


def _fin_body(acc_ref, den_ref, sw_ref, hpads_ref, pmat_ref, b_ref, out_ref):
    out_ref[...] = _combine(acc_ref, den_ref, sw_ref, hpads_ref, pmat_ref,
                            b_ref)


_prep_outs = dict(
    out_specs=[
        pl.BlockSpec((NC, R, DH), lambda i: (0, i, 0)),
        pl.BlockSpec((R, 1), lambda i: (i, 0)),
        pl.BlockSpec((R, 1), lambda i: (i, 0)),
        pl.BlockSpec((R, 1), lambda i: (i, 0)),
    ],
    out_shape=[
        jax.ShapeDtypeStruct((NC, N, DH), jnp.bfloat16),
        jax.ShapeDtypeStruct((N, 1), jnp.float32),
        jax.ShapeDtypeStruct((N, 1), jnp.float32),
        jax.ShapeDtypeStruct((N, 1), jnp.float32),
    ],
)

_acc_specs = [
    pl.BlockSpec((NC, R, DH), lambda i: (0, i, 0)),
    pl.BlockSpec((NC, R, DW), lambda i: (0, i, 0)),
    pl.BlockSpec((R, 1), lambda i: (i, 0)),
    pl.BlockSpec((NC, R, DH), lambda i: (0, i, 0)),
    pl.BlockSpec((D, D), lambda i: (0, 0)),
    pl.BlockSpec((1, D), lambda i: (0, 0)),
]


def _tc_prep(z, W, att_s, att_d):
    return pl.pallas_call(
        _prep_body,
        grid=(N // R,),
        in_specs=[
            pl.BlockSpec((R, D), lambda i: (i, 0)),
            pl.BlockSpec((D, D), lambda i: (0, 0)),
            pl.BlockSpec((D, 1), lambda i: (0, 0)),
            pl.BlockSpec((D, 1), lambda i: (0, 0)),
        ],
        **_prep_outs,
    )(z, W, att_s, att_d)


def _tc_mid(acc, den, sw, hpads, pmat, b, W, att_s, att_d):
    return pl.pallas_call(
        _mid_body,
        grid=(N // R,),
        in_specs=_acc_specs + [
            pl.BlockSpec((D, D), lambda i: (0, 0)),
            pl.BlockSpec((D, 1), lambda i: (0, 0)),
            pl.BlockSpec((D, 1), lambda i: (0, 0)),
        ],
        **_prep_outs,
    )(acc, den, sw, hpads, pmat, b, W, att_s, att_d)


def _tc_fin(acc, den, sw, hpads, pmat, b):
    return pl.pallas_call(
        _fin_body,
        grid=(N // R,),
        in_specs=_acc_specs,
        out_specs=pl.BlockSpec((R, D), lambda i: (i, 0)),
        out_shape=jax.ShapeDtypeStruct((N, D), jnp.float32),
    )(acc, den, sw, hpads, pmat, b)


def _sc_body(hpads_hbm, as_hbm, ad_hbm, src_hbm, dst_hbm, zeros_hbm, zd_hbm,
             acc_out, den_out,
             srcv, dstv, astab, adtab, wbufs, rows_bf, rows_f, acc_sp, den_sp,
             gsem, ssem, wsem, psem):
    c = lax.axis_index("c")
    s = lax.axis_index("s")

    pltpu.async_copy(src_hbm.at[s], srcv, psem)
    pltpu.async_copy(dst_hbm.at[s], dstv, psem)
    pltpu.async_copy(as_hbm, astab, psem)
    pltpu.async_copy(ad_hbm, adtab, psem)
    pltpu.async_copy(zeros_hbm, acc_sp.at[pl.ds(s * RSUB, RSUB)], psem)
    pltpu.async_copy(zd_hbm.at[pl.ds(0, RSUB)],
                     den_sp.at[pl.ds(s * RSUB, RSUB)], psem)
    pltpu.async_copy(zd_hbm.at[pl.ds(0, NBW * K)], wbufs, psem)

    @pl.when(s == NS - 1)
    def _():
        pltpu.async_copy(zeros_hbm.at[pl.ds(0, RTAIL)],
                         acc_sp.at[pl.ds(NS * RSUB, RTAIL)], psem)
        pltpu.async_copy(zd_hbm.at[pl.ds(0, RTAIL)],
                         den_sp.at[pl.ds(NS * RSUB, RTAIL)], psem)

    pltpu.make_async_copy(src_hbm.at[s], srcv, psem).wait()
    pltpu.make_async_copy(dst_hbm.at[s], dstv, psem).wait()
    pltpu.make_async_copy(as_hbm, astab, psem).wait()
    pltpu.make_async_copy(ad_hbm, adtab, psem).wait()
    pltpu.make_async_copy(zeros_hbm, acc_sp.at[pl.ds(s * RSUB, RSUB)],
                          psem).wait()
    pltpu.make_async_copy(zd_hbm.at[pl.ds(0, RSUB)],
                          den_sp.at[pl.ds(s * RSUB, RSUB)], psem).wait()
    pltpu.make_async_copy(zd_hbm.at[pl.ds(0, NBW * K)], wbufs, psem).wait()

    @pl.when(s == NS - 1)
    def _():
        pltpu.make_async_copy(zeros_hbm.at[pl.ds(0, RTAIL)],
                              acc_sp.at[pl.ds(NS * RSUB, RTAIL)], psem).wait()
        pltpu.make_async_copy(zd_hbm.at[pl.ds(0, RTAIL)],
                              den_sp.at[pl.ds(NS * RSUB, RTAIL)], psem).wait()

    plsc.subcore_barrier()
    htab = hpads_hbm.at[c]
    col0 = jnp.zeros((16,), jnp.int32)
    lane = lax.iota(jnp.int32, 16)

    pltpu.async_copy(htab.at[srcv.at[0]], rows_bf.at[0], gsem)

    def chunk(i, carry):
        bg = lax.rem(i, NBG)
        bs = lax.rem(i, NBS)
        bw = lax.rem(i, NBW)

        @pl.when(i >= NBW)
        def _():
            pltpu.make_async_copy(
                wbufs.at[pl.ds(bw * K, K)],
                den_sp.at[dstv.at[i - NBW]], wsem).wait()

        for j in range(K // 16):
            sidx = srcv[i, pl.ds(j * 16, 16)]
            didx = dstv[i, pl.ds(j * 16, 16)]
            a = plsc.load_gather(astab, [sidx]) + plsc.load_gather(adtab, [didx])
            w16 = jnp.exp(jnp.maximum(a, 0.2 * a))
            plsc.store_scatter(wbufs, [bw * K + j * 16 + lane, col0], w16)
        pltpu.async_copy(wbufs.at[pl.ds(bw * K, K)],
                         den_sp.at[dstv.at[i]], wsem, add=True)

        @pl.when(i < NCHUNK - 1)
        def _():
            pltpu.async_copy(htab.at[srcv.at[i + 1]], rows_bf.at[1 - bg], gsem)

        @pl.when(i >= NBS)
        def _():
            pltpu.make_async_copy(rows_f.at[bs],
                                  acc_sp.at[dstv.at[i - NBS]], ssem).wait()

        pltpu.make_async_copy(htab.at[srcv.at[i]], rows_bf.at[bg], gsem).wait()

        @plsc.parallel_loop(0, K, unroll=8)
        def scale(k):
            wv = plsc.load_gather(wbufs, [jnp.full((16,), bw * K + k,
                                                   jnp.int32), col0])
            for g in range(DH // 32):
                v = rows_bf[bg, k, pl.ds(g * 32, 32)]
                ev, od = plsc.unpack(v, format=plsc.PackFormat.INTERLEAVED,
                                     preferred_element_type=jnp.float32)
                rows_f[bs, k, pl.ds(g * 32, 16)] = ev * wv
                rows_f[bs, k, pl.ds(g * 32 + 16, 16)] = od * wv

        pltpu.async_copy(rows_f.at[bs], acc_sp.at[dstv.at[i]], ssem, add=True)
        return carry

    lax.fori_loop(0, NCHUNK, chunk, 0)
    for t in range(NBS):
        i = NCHUNK - NBS + t
        pltpu.make_async_copy(rows_f.at[lax.rem(i, NBS)],
                              acc_sp.at[dstv.at[i]], ssem).wait()
    for t in range(NBW):
        i = NCHUNK - NBW + t
        pltpu.make_async_copy(wbufs.at[pl.ds(lax.rem(i, NBW) * K, K)],
                              den_sp.at[dstv.at[i]], wsem).wait()
    plsc.subcore_barrier()
    pltpu.sync_copy(acc_sp.at[pl.ds(s * RSUB, RSUB)],
                    acc_out.at[c, pl.ds(s * RSUB, RSUB)])
    pltpu.sync_copy(den_sp.at[pl.ds(s * RSUB, RSUB)],
                    den_out.at[c, pl.ds(s * RSUB, RSUB)])

    @pl.when(s == NS - 1)
    def _():
        pltpu.sync_copy(acc_sp.at[pl.ds(NS * RSUB, RTAIL)],
                        acc_out.at[c, pl.ds(NS * RSUB, RTAIL)])
        pltpu.sync_copy(den_sp.at[pl.ds(NS * RSUB, RTAIL)],
                        den_out.at[c, pl.ds(NS * RSUB, RTAIL)])


def _sc_edge(hpads, a_s, a_d, src, dst, zeros_blk, zd):
    mesh = plsc.VectorSubcoreMesh(core_axis_name="c", subcore_axis_name="s",
                                  num_cores=NC, num_subcores=NS)
    f = pl.kernel(
        _sc_body,
        out_type=[
            jax.ShapeDtypeStruct((NC, N, DH), jnp.float32),
            jax.ShapeDtypeStruct((NC, N, DW), jnp.float32),
        ],
        mesh=mesh,
        compiler_params=pltpu.CompilerParams(needs_layout_passes=False,
                                             use_tc_tiling_on_sc=False),
        scratch_types=[
            pltpu.VMEM((NCHUNK, K), jnp.int32),
            pltpu.VMEM((NCHUNK, K), jnp.int32),
            pltpu.VMEM((N,), jnp.float32),
            pltpu.VMEM((N,), jnp.float32),
            pltpu.VMEM((NBW * K, DW), jnp.float32),
            pltpu.VMEM((NBG, K, DH), jnp.bfloat16),
            pltpu.VMEM((NBS, K, DH), jnp.float32),
            pltpu.VMEM_SHARED((N, DH), jnp.float32),
            pltpu.VMEM_SHARED((N, DW), jnp.float32),
            pltpu.SemaphoreType.DMA,
            pltpu.SemaphoreType.DMA,
            pltpu.SemaphoreType.DMA,
            pltpu.SemaphoreType.DMA,
        ],
    )
    return f(hpads, a_s, a_d, src, dst, zeros_blk, zd)


def kernel(x, edge_index, W1, att_src1, att_dst1, b1, W2, att_src2, att_dst2,
           b2):
    src = edge_index[0].reshape(NS, NCHUNK, K)
    dst = edge_index[1].reshape(NS, NCHUNK, K)
    zeros_blk = jnp.zeros((RSUB, DH), jnp.float32)
    zd = jnp.zeros((RSUB, DW), jnp.float32)
    pmat = jnp.eye(D, dtype=jnp.float32)[PERM]

    hpads1, a_s1, a_d1, sw1 = _tc_prep(
        x, W1[:, PERM], att_src1.reshape(D, 1)[PERM],
        att_dst1.reshape(D, 1)[PERM])
    acc1, den1 = _sc_edge(hpads1, a_s1.reshape(N), a_d1.reshape(N), src, dst,
                          zeros_blk, zd)
    hpads2, a_s2, a_d2, sw2 = _tc_mid(
        acc1, den1, sw1, hpads1, pmat, b1.reshape(1, D), W2[:, PERM],
        att_src2.reshape(D, 1)[PERM], att_dst2.reshape(D, 1)[PERM])
    acc2, den2 = _sc_edge(hpads2, a_s2.reshape(N), a_d2.reshape(N), src, dst,
                          zeros_blk, zd)
    return _tc_fin(acc2, den2, sw2, hpads2, pmat, b2.reshape(1, D))
